# baseline (speedup 1.0000x reference)
import functools
import jax
import jax.numpy as jnp
from jax import lax
from jax.experimental import pallas as pl
from jax.experimental.pallas import tpu as pltpu
from jax.experimental.pallas import tpu_sc as plsc

_DEPTH = 1000
_ROWS = 4096
_COLS = 26
_NW = 32
_RPW = _ROWS // _NW
_CH = 2
_NCHUNK = _RPW // _CH


@functools.partial(
    pl.kernel,
    out_type=jax.ShapeDtypeStruct((_ROWS, _COLS, _DEPTH), jnp.float32),
    mesh=plsc.VectorSubcoreMesh(core_axis_name="c", subcore_axis_name="s"),
    compiler_params=pltpu.CompilerParams(
        use_tc_tiling_on_sc=False, needs_layout_passes=False),
    scratch_types=[
        pltpu.VMEM((_RPW * _COLS,), jnp.int32),
        pltpu.VMEM((_CH, _COLS, _DEPTH), jnp.float32),
        pltpu.VMEM((_CH, _COLS, _DEPTH), jnp.float32),
        pltpu.SemaphoreType.DMA,
        pltpu.SemaphoreType.DMA,
    ],
)
def _sc_body(ids_hbm, out_hbm, ids_v, buf0, buf1, sem0, sem1):
    wid = lax.axis_index("s") * 2 + lax.axis_index("c")
    row0 = wid * _RPW
    pltpu.sync_copy(ids_hbm.at[pl.ds(row0 * _COLS, _RPW * _COLS)], ids_v)

    bufs = (buf0, buf1)
    sems = (sem0, sem1)
    iota = lax.iota(jnp.int32, 16)
    ones = jnp.full((16,), 1.0, jnp.float32)
    zeros = jnp.zeros((16,), jnp.float32)

    def _zinit(j, carry):
        for r in range(_CH):
            for buf in bufs:
                for k in range(62):
                    buf[r, j, pl.ds(k * 16, 16)] = zeros
                buf[r, j, pl.ds(_DEPTH - 16, 16)] = zeros
        return carry

    lax.fori_loop(0, _COLS, _zinit, 0)

    def _scatter(buf, c, val):
        for r in range(_CH):
            off = (c * _CH + r) * _COLS
            ca = ids_v[pl.ds(off, 16)]
            cb = ids_v[pl.ds(off + (_COLS - 16), 16)]
            rr = jnp.full((16,), r, jnp.int32)
            plsc.store_scatter(buf, [rr, iota, ca], val)
            plsc.store_scatter(buf, [rr, iota + (_COLS - 16), cb], val)

    def _dma(buf, sem, c):
        return pltpu.make_async_copy(
            buf, out_hbm.at[pl.ds(row0 + c * _CH, _CH)], sem)

    for b in range(2):
        _scatter(bufs[b], b, ones)
        _dma(bufs[b], sems[b], b).start()

    def _pair(p, carry):
        for b in range(2):
            c = p * 2 + b
            _dma(bufs[b], sems[b], c - 2).wait()
            _scatter(bufs[b], c - 2, zeros)
            _scatter(bufs[b], c, ones)
            _dma(bufs[b], sems[b], c).start()
        return carry

    lax.fori_loop(1, _NCHUNK // 2, _pair, 0)

    for b in range(2):
        _dma(bufs[b], sems[b], _NCHUNK - 2 + b).wait()


def kernel(inputs):
    ids = inputs.astype(jnp.int32).reshape(-1)
    return _sc_body(ids)


# speedup vs baseline: 1.9708x; 1.9708x over previous
import functools
import jax
import jax.numpy as jnp
from jax import lax
from jax.experimental import pallas as pl
from jax.experimental.pallas import tpu as pltpu
from jax.experimental.pallas import tpu_sc as plsc

_DEPTH = 1000
_ROWS = 4096
_COLS = 26
_NW = 32
_RPW = _ROWS // _NW
_CH = 1
_NCHUNK = _RPW // _CH


@functools.partial(
    pl.kernel,
    out_type=jax.ShapeDtypeStruct((_ROWS, _COLS, _DEPTH), jnp.float32),
    mesh=plsc.VectorSubcoreMesh(core_axis_name="c", subcore_axis_name="s"),
    compiler_params=pltpu.CompilerParams(
        use_tc_tiling_on_sc=True, needs_layout_passes=False),
    scratch_types=[
        pltpu.VMEM((_RPW * _COLS,), jnp.int32),
        pltpu.VMEM((_CH, _COLS, _DEPTH), jnp.float32),
        pltpu.VMEM((_CH, _COLS, _DEPTH), jnp.float32),
        pltpu.SemaphoreType.DMA,
        pltpu.SemaphoreType.DMA,
    ],
)
def _sc_body(ids_hbm, out_hbm, ids_v, buf0, buf1, sem0, sem1):
    wid = lax.axis_index("s") * 2 + lax.axis_index("c")
    row0 = wid * _RPW
    pltpu.sync_copy(ids_hbm.at[pl.ds(row0 * _COLS, _RPW * _COLS)], ids_v)

    bufs = (buf0, buf1)
    sems = (sem0, sem1)
    iota = lax.iota(jnp.int32, 16)
    ones = jnp.full((16,), 1.0, jnp.float32)
    zeros = jnp.zeros((16,), jnp.float32)

    def _zinit(j, carry):
        for r in range(_CH):
            for buf in bufs:
                for k in range(62):
                    buf[r, j, pl.ds(k * 16, 16)] = zeros
                buf[r, j, pl.ds(_DEPTH - 16, 16)] = zeros
        return carry

    lax.fori_loop(0, _COLS, _zinit, 0)

    def _scatter(buf, c, val):
        for r in range(_CH):
            off = (c * _CH + r) * _COLS
            ca = ids_v[pl.ds(off, 16)]
            cb = ids_v[pl.ds(off + (_COLS - 16), 16)]
            rr = jnp.full((16,), r, jnp.int32)
            plsc.store_scatter(buf, [rr, iota, ca], val)
            plsc.store_scatter(buf, [rr, iota + (_COLS - 16), cb], val)

    def _dma(buf, sem, c):
        return pltpu.make_async_copy(
            buf, out_hbm.at[pl.ds(row0 + c * _CH, _CH)], sem)

    for b in range(2):
        _scatter(bufs[b], b, ones)
        _dma(bufs[b], sems[b], b).start()

    def _pair(p, carry):
        for b in range(2):
            c = p * 2 + b
            _dma(bufs[b], sems[b], c - 2).wait()
            _scatter(bufs[b], c - 2, zeros)
            _scatter(bufs[b], c, ones)
            _dma(bufs[b], sems[b], c).start()
        return carry

    lax.fori_loop(1, _NCHUNK // 2, _pair, 0)

    for b in range(2):
        _dma(bufs[b], sems[b], _NCHUNK - 2 + b).wait()


def kernel(inputs):
    ids = inputs.astype(jnp.int32).reshape(-1)
    return _sc_body(ids)


# speedup vs baseline: 1.9720x; 1.0006x over previous
import functools
import jax
import jax.numpy as jnp
from jax import lax
from jax.experimental import pallas as pl
from jax.experimental.pallas import tpu as pltpu
from jax.experimental.pallas import tpu_sc as plsc

_DEPTH = 1000
_ROWS = 4096
_COLS = 26
_NW = 32
_RPW = _ROWS // _NW
_CH = 1
_NCHUNK = _RPW // _CH


@functools.partial(
    pl.kernel,
    out_type=jax.ShapeDtypeStruct((_ROWS, _COLS, _DEPTH), jnp.float32),
    mesh=plsc.VectorSubcoreMesh(core_axis_name="c", subcore_axis_name="s"),
    compiler_params=pltpu.CompilerParams(
        use_tc_tiling_on_sc=True, needs_layout_passes=False,
        skip_device_barrier=True),
    scratch_types=[
        pltpu.VMEM((_RPW * _COLS,), jnp.int32),
        pltpu.VMEM((_CH, _COLS, _DEPTH), jnp.float32),
        pltpu.VMEM((_CH, _COLS, _DEPTH), jnp.float32),
        pltpu.SemaphoreType.DMA,
        pltpu.SemaphoreType.DMA,
    ],
)
def _sc_body(ids_hbm, out_hbm, ids_v, buf0, buf1, sem0, sem1):
    wid = lax.axis_index("s") * 2 + lax.axis_index("c")
    row0 = wid * _RPW
    pltpu.sync_copy(ids_hbm.at[pl.ds(row0 * _COLS, _RPW * _COLS)], ids_v)

    bufs = (buf0, buf1)
    sems = (sem0, sem1)
    iota = lax.iota(jnp.int32, 16)
    ones = jnp.full((16,), 1.0, jnp.float32)
    zeros = jnp.zeros((16,), jnp.float32)

    def _zinit(j, carry):
        for r in range(_CH):
            for buf in bufs:
                for k in range(62):
                    buf[r, j, pl.ds(k * 16, 16)] = zeros
                buf[r, j, pl.ds(_DEPTH - 16, 16)] = zeros
        return carry

    lax.fori_loop(0, _COLS, _zinit, 0)

    def _scatter(buf, c, val):
        for r in range(_CH):
            off = (c * _CH + r) * _COLS
            ca = ids_v[pl.ds(off, 16)]
            cb = ids_v[pl.ds(off + (_COLS - 16), 16)]
            rr = jnp.full((16,), r, jnp.int32)
            plsc.store_scatter(buf, [rr, iota, ca], val)
            plsc.store_scatter(buf, [rr, iota + (_COLS - 16), cb], val)

    def _dma(buf, sem, c):
        return pltpu.make_async_copy(
            buf, out_hbm.at[pl.ds(row0 + c * _CH, _CH)], sem)

    for b in range(2):
        _scatter(bufs[b], b, ones)
        _dma(bufs[b], sems[b], b).start()

    def _pair(p, carry):
        for b in range(2):
            c = p * 2 + b
            _dma(bufs[b], sems[b], c - 2).wait()
            _scatter(bufs[b], c - 2, zeros)
            _scatter(bufs[b], c, ones)
            _dma(bufs[b], sems[b], c).start()
        return carry

    lax.fori_loop(1, _NCHUNK // 2, _pair, 0)

    for b in range(2):
        _dma(bufs[b], sems[b], _NCHUNK - 2 + b).wait()


def kernel(inputs):
    ids = inputs.astype(jnp.int32).reshape(-1)
    return _sc_body(ids)
